# baseline (speedup 1.0000x reference)
import functools

import jax
import jax.numpy as jnp
from jax import lax
from jax.experimental import pallas as pl
from jax.experimental.pallas import tpu as pltpu
from jax.experimental.pallas import tpu_sc as plsc

VOCAB = 1000000
EMBED = 32
BATCH = 4096
SEQ = 200

ROW = 128
NROWS = BATCH * SEQ // ROW
NW = 32
ROWS_PER_W = NROWS // NW

NBLK = VOCAB // ROW
BLK_TAIL = VOCAB - NBLK * ROW
NTJ = NBLK + 1
NTILES = 4 * NTJ
LIN_ROWS = VOCAB * EMBED // ROW

_MESH = dict(core_axis_name="c", subcore_axis_name="s")


def _wid():
  return lax.axis_index("s") * 2 + lax.axis_index("c")


def _make_a1():
  @functools.partial(
      pl.kernel,
      mesh=plsc.VectorSubcoreMesh(**_MESH),
      compiler_params=pltpu.CompilerParams(use_tc_tiling_on_sc=True),
      out_type=jax.ShapeDtypeStruct((NTILES, 8, ROW), jnp.float32),
      scratch_types=[pltpu.SemaphoreType.DMA],
  )
  def body(tt_hbm, raw_hbm, sem):
    wid = _wid()
    nfull = 4 * NBLK
    cnt = nfull // NW + (wid < nfull % NW)
    K = 8

    def one(f):
      et = f // NBLK
      c = f % NBLK
      r = f + et
      return pltpu.async_copy(
          tt_hbm.at[pl.ds(et * 8, 8), pl.ds(c * ROW, ROW)],
          raw_hbm.at[r],
          sem,
      )

    def step(t, carry):
      copies = [one(wid + NW * (K * t + k)) for k in range(K)]
      for cp in copies:
        cp.wait()
      return carry

    lax.fori_loop(0, cnt // K, step, 0)

    def step1(t, carry):
      one(wid + NW * t).wait()
      return carry

    lax.fori_loop((cnt // K) * K, cnt, step1, 0)

  return body


def _transpose_into(dst, src, a_dim, n_vregs):
  iota = lax.iota(jnp.int32, 16)
  n_a = a_dim // 16
  a_vecs = [iota + (h * 16) for h in range(n_a)]
  for j in range(n_vregs):
    a_vec = a_vecs[j % n_a]
    b0 = j // n_a
    b_vec = jnp.full((16,), b0, jnp.int32)
    v = plsc.load_gather(src, [a_vec, b_vec])
    dst[j // 8, pl.ds((j % 8) * 16, 16)] = v


def _make_a2():
  @functools.partial(
      pl.kernel,
      mesh=plsc.VectorSubcoreMesh(**_MESH),
      compiler_params=pltpu.CompilerParams(
          use_tc_tiling_on_sc=False, needs_layout_passes=False
      ),
      out_type=jax.ShapeDtypeStruct((LIN_ROWS, ROW), jnp.float32),
      scratch_types=[
          pltpu.VMEM((EMBED, ROW), jnp.float32),
          pltpu.VMEM((EMBED, ROW), jnp.float32),
          pltpu.SemaphoreType.DMA,
      ],
  )
  def body(raw_hbm, tail_hbm, lin_hbm, inb, outb, sem):
    wid = _wid()
    cnt = NBLK // NW + (wid < NBLK % NW)

    def step(t, carry):
      c = wid + NW * t
      for et in range(4):
        pltpu.async_copy(
            raw_hbm.at[et * NTJ + c], inb.at[pl.ds(et * 8, 8)], sem
        )
      pltpu.make_async_copy(raw_hbm.at[0], inb.at[pl.ds(0, 8)], sem).wait()
      pltpu.make_async_copy(raw_hbm.at[0], inb.at[pl.ds(0, 8)], sem).wait()
      pltpu.make_async_copy(raw_hbm.at[0], inb.at[pl.ds(0, 8)], sem).wait()
      pltpu.make_async_copy(raw_hbm.at[0], inb.at[pl.ds(0, 8)], sem).wait()
      _transpose_into(outb, inb, a_dim=EMBED, n_vregs=256)
      pltpu.sync_copy(outb, lin_hbm.at[pl.ds(c * EMBED, EMBED)])
      return carry

    lax.fori_loop(0, cnt, step, 0)

    @pl.when(wid == 17)
    def _tail():
      pltpu.sync_copy(tail_hbm, lin_hbm.at[pl.ds(NBLK * EMBED, 16)])

  return body


def _make_phase_b():
  @functools.partial(
      pl.kernel,
      mesh=plsc.VectorSubcoreMesh(**_MESH),
      compiler_params=pltpu.CompilerParams(
          use_tc_tiling_on_sc=False, needs_layout_passes=False
      ),
      out_type=jax.ShapeDtypeStruct((NROWS * 4, 8, ROW), jnp.float32),
      scratch_types=[
          pltpu.VMEM((ROWS_PER_W, ROW), jnp.int32),
          pltpu.VMEM((ROW, EMBED), jnp.float32),
          pltpu.VMEM((EMBED, ROW), jnp.float32),
          pltpu.SemaphoreType.DMA,
      ],
  )
  def body(idx_hbm, tab_hbm, out_hbm, idx_all, rows, trb, sem):
    wid = _wid()
    q0 = wid * ROWS_PER_W
    pltpu.sync_copy(idx_hbm.at[pl.ds(q0, ROWS_PER_W)], idx_all)

    def step(n, carry):
      q = q0 + n
      pltpu.async_copy(tab_hbm.at[idx_all.at[n]], rows, sem).wait()
      _transpose_into(trb, rows, a_dim=ROW, n_vregs=256)
      st = q // 256
      r = q % 256
      bt = r // 8
      si = r % 8
      s = st * 8 + si
      for et in range(4):
        blk = (s * 4 + et) * 32 + bt
        pltpu.sync_copy(trb.at[pl.ds(et * 8, 8)], out_hbm.at[blk])
      return carry

    lax.fori_loop(0, ROWS_PER_W, step, 0)

  return body


_a1 = _make_a1()
_a2 = _make_a2()
_phase_b = _make_phase_b()


def kernel(words_seq, table):
  raw = _a1(table.T)
  tail = table[NBLK * ROW :, :].reshape(16, ROW)
  tab = _a2(raw, tail).reshape(VOCAB, EMBED)
  ws = words_seq.astype(jnp.int32)
  idx = ws.T.reshape(25, 8, 32, 128).transpose(0, 2, 1, 3).reshape(NROWS, ROW)
  out = _phase_b(idx, tab)
  out5 = out.reshape(SEQ, 4, 32, 8, ROW)
  return out5.transpose(2, 4, 0, 1, 3).reshape(BATCH, SEQ, EMBED)


# speedup vs baseline: 1.0621x; 1.0621x over previous
import functools

import jax
import jax.numpy as jnp
from jax import lax
from jax.experimental import pallas as pl
from jax.experimental.pallas import tpu as pltpu
from jax.experimental.pallas import tpu_sc as plsc

VOCAB = 1000000
EMBED = 32
BATCH = 4096
SEQ = 200

ROW = 128
NROWS = BATCH * SEQ // ROW
NW = 32
ROWS_PER_W = NROWS // NW

NBLK = VOCAB // ROW
NTJ = NBLK + 1
VPAD = NTJ * ROW
LIN_ROWS = VOCAB * EMBED // ROW

_MESH = dict(core_axis_name="c", subcore_axis_name="s")


def _wid():
  return lax.axis_index("s") * 2 + lax.axis_index("c")


def _make_a1():
  @functools.partial(
      pl.kernel,
      mesh=plsc.VectorSubcoreMesh(**_MESH),
      compiler_params=pltpu.CompilerParams(use_tc_tiling_on_sc=True),
      out_type=jax.ShapeDtypeStruct((4, 8, VPAD), jnp.float32),
  )
  def body(tt_hbm, raw_hbm):
    wid = _wid()
    et = wid // 8
    k = wid % 8
    W0 = 977 * ROW
    W1 = 976 * ROW
    x0 = jnp.where(k < 4, k * W0, 4 * W0 + (k - 4) * W1)

    @pl.when(k < 4)
    def _():
      pltpu.sync_copy(
          tt_hbm.at[pl.ds(et * 8, 8), pl.ds(x0, W0)],
          raw_hbm.at[et, :, pl.ds(x0, W0)],
      )

    @pl.when(k >= 4)
    def _():
      pltpu.sync_copy(
          tt_hbm.at[pl.ds(et * 8, 8), pl.ds(x0, W1)],
          raw_hbm.at[et, :, pl.ds(x0, W1)],
      )

  return body


def _transpose_into(dst, src, a_dim, n_vregs):
  iota = lax.iota(jnp.int32, 16)
  n_a = a_dim // 16
  a_vecs = [iota + (h * 16) for h in range(n_a)]
  nd = len(dst.shape)
  for j in range(n_vregs):
    a_vec = a_vecs[j % n_a]
    b0 = j // n_a
    b_vec = jnp.full((16,), b0, jnp.int32)
    v = plsc.load_gather(src, [a_vec, b_vec])
    flat = 16 * j
    if nd == 2:
      dst[flat // 128, pl.ds(flat % 128, 16)] = v
    else:
      dst[flat // 1024, (flat % 1024) // 128, pl.ds(flat % 128, 16)] = v


def _make_a2():
  @functools.partial(
      pl.kernel,
      mesh=plsc.VectorSubcoreMesh(**_MESH),
      compiler_params=pltpu.CompilerParams(
          use_tc_tiling_on_sc=False, needs_layout_passes=False
      ),
      out_type=jax.ShapeDtypeStruct((LIN_ROWS, ROW), jnp.float32),
      scratch_types=[
          pltpu.VMEM((EMBED, ROW), jnp.float32),
          pltpu.VMEM((EMBED, ROW), jnp.float32),
          pltpu.VMEM((EMBED, ROW), jnp.float32),
          pltpu.VMEM((EMBED, ROW), jnp.float32),
          pltpu.SemaphoreType.DMA,
          pltpu.SemaphoreType.DMA,
          pltpu.SemaphoreType.DMA,
          pltpu.SemaphoreType.DMA,
      ],
  )
  def body(raw_hbm, tail_hbm, lin_hbm, ina, inb, outa, outb,
           gsa, gsb, wsa, wsb):
    wid = _wid()
    c0 = jnp.where(wid < 4, wid * 245, 980 + (wid - 4) * 244)

    def fire_in(c, buf, sem):
      for et in range(4):
        pltpu.async_copy(raw_hbm.at[et * NTJ + c], buf.at[pl.ds(et * 8, 8)],
                         sem)

    def wait_in(buf, sem):
      pltpu.make_async_copy(raw_hbm.at[0], buf.at[pl.ds(0, 8)], sem).wait()
      pltpu.make_async_copy(raw_hbm.at[0], buf.at[pl.ds(0, 8)], sem).wait()
      pltpu.make_async_copy(raw_hbm.at[0], buf.at[pl.ds(0, 8)], sem).wait()
      pltpu.make_async_copy(raw_hbm.at[0], buf.at[pl.ds(0, 8)], sem).wait()

    def fire_out(c, buf, sem):
      pltpu.async_copy(buf, lin_hbm.at[pl.ds(c * EMBED, EMBED)], sem)

    def wait_out(buf, sem):
      pltpu.make_async_copy(buf, lin_hbm.at[pl.ds(0, EMBED)], sem).wait()

    fire_in(c0, ina, gsa)
    fire_in(c0 + 1, inb, gsb)
    wait_in(ina, gsa)
    _transpose_into(outa, ina, a_dim=EMBED, n_vregs=256)
    fire_in(c0 + 2, ina, gsa)
    fire_out(c0, outa, wsa)
    wait_in(inb, gsb)
    _transpose_into(outb, inb, a_dim=EMBED, n_vregs=256)
    fire_in(c0 + 3, inb, gsb)
    fire_out(c0 + 1, outb, wsb)

    def step(m, carry):
      ca = c0 + 2 * m
      cb = ca + 1
      wait_in(ina, gsa)
      wait_out(outa, wsa)
      _transpose_into(outa, ina, a_dim=EMBED, n_vregs=256)
      fire_in(ca + 2, ina, gsa)
      fire_out(ca, outa, wsa)
      wait_in(inb, gsb)
      wait_out(outb, wsb)
      _transpose_into(outb, inb, a_dim=EMBED, n_vregs=256)
      fire_in(cb + 2, inb, gsb)
      fire_out(cb, outb, wsb)
      return carry

    lax.fori_loop(1, 121, step, 0)

    wait_in(ina, gsa)
    wait_out(outa, wsa)
    _transpose_into(outa, ina, a_dim=EMBED, n_vregs=256)
    fire_out(c0 + 242, outa, wsa)
    wait_in(inb, gsb)
    wait_out(outb, wsb)
    _transpose_into(outb, inb, a_dim=EMBED, n_vregs=256)
    fire_out(c0 + 243, outb, wsb)
    wait_out(outa, wsa)
    wait_out(outb, wsb)

    @pl.when(wid < 4)
    def _extra():
      fire_in(c0 + 244, ina, gsa)
      wait_in(ina, gsa)
      _transpose_into(outa, ina, a_dim=EMBED, n_vregs=256)
      pltpu.sync_copy(outa, lin_hbm.at[pl.ds((c0 + 244) * EMBED, EMBED)])

    @pl.when(wid == 17)
    def _tail():
      pltpu.sync_copy(tail_hbm, lin_hbm.at[pl.ds(NBLK * EMBED, 16)])

  return body


def _make_phase_b():
  @functools.partial(
      pl.kernel,
      mesh=plsc.VectorSubcoreMesh(**_MESH),
      compiler_params=pltpu.CompilerParams(
          use_tc_tiling_on_sc=False, needs_layout_passes=False
      ),
      out_type=jax.ShapeDtypeStruct((NROWS * 4, 8, ROW), jnp.float32),
      scratch_types=[
          pltpu.VMEM((ROWS_PER_W, ROW), jnp.int32),
          pltpu.VMEM((ROW, EMBED), jnp.float32),
          pltpu.VMEM((ROW, EMBED), jnp.float32),
          pltpu.VMEM((4, 8, ROW), jnp.float32),
          pltpu.VMEM((4, 8, ROW), jnp.float32),
          pltpu.SemaphoreType.DMA,
          pltpu.SemaphoreType.DMA,
          pltpu.SemaphoreType.DMA,
          pltpu.SemaphoreType.DMA,
      ],
  )
  def body(idx_hbm, tab_hbm, out_hbm, idx_all, rowsa, rowsb, tra, trb,
           gsa, gsb, wsa, wsb):
    wid = _wid()
    q0 = wid * ROWS_PER_W
    pltpu.sync_copy(idx_hbm.at[pl.ds(q0, ROWS_PER_W)], idx_all)

    def fire_g(n, buf, sem):
      pltpu.async_copy(tab_hbm.at[idx_all.at[n]], buf, sem)

    def wait_g(buf, sem):
      pltpu.make_async_copy(tab_hbm.at[pl.ds(0, ROW)], buf, sem).wait()

    def fire_w(n, buf, sem):
      q = q0 + n
      st = q // 256
      r = q % 256
      bt = r // 8
      si = r % 8
      s = st * 8 + si
      for et in range(4):
        pltpu.async_copy(buf.at[et], out_hbm.at[(s * 4 + et) * 32 + bt], sem)

    def wait_w(buf, sem):
      pltpu.make_async_copy(buf, out_hbm.at[pl.ds(0, 4)], sem).wait()

    fire_g(0, rowsa, gsa)
    fire_g(1, rowsb, gsb)
    wait_g(rowsa, gsa)
    _transpose_into(tra, rowsa, a_dim=ROW, n_vregs=256)
    fire_g(2, rowsa, gsa)
    fire_w(0, tra, wsa)
    wait_g(rowsb, gsb)
    _transpose_into(trb, rowsb, a_dim=ROW, n_vregs=256)
    fire_g(3, rowsb, gsb)
    fire_w(1, trb, wsb)

    def step(m, carry):
      na = 2 * m
      nb = na + 1
      wait_g(rowsa, gsa)
      wait_w(tra, wsa)
      _transpose_into(tra, rowsa, a_dim=ROW, n_vregs=256)
      fire_g(na + 2, rowsa, gsa)
      fire_w(na, tra, wsa)
      wait_g(rowsb, gsb)
      wait_w(trb, wsb)
      _transpose_into(trb, rowsb, a_dim=ROW, n_vregs=256)
      fire_g(nb + 2, rowsb, gsb)
      fire_w(nb, trb, wsb)
      return carry

    lax.fori_loop(1, 99, step, 0)

    wait_g(rowsa, gsa)
    wait_w(tra, wsa)
    _transpose_into(tra, rowsa, a_dim=ROW, n_vregs=256)
    fire_w(198, tra, wsa)
    wait_g(rowsb, gsb)
    wait_w(trb, wsb)
    _transpose_into(trb, rowsb, a_dim=ROW, n_vregs=256)
    fire_w(199, trb, wsb)
    wait_w(tra, wsa)
    wait_w(trb, wsb)

  return body


_a1 = _make_a1()
_a2 = _make_a2()
_phase_b = _make_phase_b()


def kernel(words_seq, table):
  raw = _a1(table.T)
  raw4 = raw.reshape(4, 8, NTJ, ROW).transpose(0, 2, 1, 3)
  tail = table[NBLK * ROW :, :].reshape(16, ROW)
  tab = _a2(raw4.reshape(4 * NTJ, 8, ROW), tail).reshape(VOCAB, EMBED)
  ws = words_seq.astype(jnp.int32)
  idx = ws.T.reshape(25, 8, 32, 128).transpose(0, 2, 1, 3).reshape(NROWS, ROW)
  out = _phase_b(idx, tab)
  out5 = out.reshape(SEQ, 4, 32, 8, ROW)
  return out5.transpose(2, 4, 0, 1, 3).reshape(BATCH, SEQ, EMBED)
